# Initial kernel scaffold; baseline (speedup 1.0000x reference)
import functools

import jax
import jax.numpy as jnp
from jax import lax
from jax.experimental import pallas as pl
from jax.experimental.pallas import tpu as pltpu
from jax.experimental.pallas import tpu_sc as plsc

_NUM_CORES = 2
_NUM_SUBCORES = 16
_NW = _NUM_CORES * _NUM_SUBCORES

_CHUNK = 128
_D = 300
_B = 4096 * 50
_PER_W = _B // _NW
_NCH = _PER_W // _CHUNK


def _make_gather():
  mesh = plsc.VectorSubcoreMesh(core_axis_name="c", subcore_axis_name="s")

  @functools.partial(
      pl.kernel,
      mesh=mesh,
      out_type=jax.ShapeDtypeStruct((_B, _D), jnp.float32),
      scratch_types=[
          pltpu.VMEM((_NCH, _CHUNK), jnp.int32),
          pltpu.VMEM((_CHUNK, _D), jnp.float32),
          pltpu.VMEM((_CHUNK, _D), jnp.float32),
          pltpu.SemaphoreType.DMA,
          pltpu.SemaphoreType.DMA,
      ],
  )
  def gather_kernel(idx_hbm, table_hbm, out_hbm, idx_v, buf0, buf1,
                    sem0, sem1):
    wid = lax.axis_index("s") * _NUM_CORES + lax.axis_index("c")
    base = wid * _PER_W

    pltpu.sync_copy(idx_hbm.at[wid], idx_v)

    pltpu.make_async_copy(table_hbm.at[idx_v.at[0]], buf0, sem0).start()
    pltpu.make_async_copy(table_hbm.at[idx_v.at[1]], buf1, sem1).start()

    def body(i, _):
      j0 = 2 * i
      j1 = 2 * i + 1

      pltpu.make_async_copy(table_hbm.at[idx_v.at[j0]], buf0, sem0).wait()
      pltpu.sync_copy(buf0, out_hbm.at[pl.ds(base + j0 * _CHUNK, _CHUNK)])

      @pl.when(j0 + 2 < _NCH)
      def _():
        pltpu.make_async_copy(
            table_hbm.at[idx_v.at[j0 + 2]], buf0, sem0).start()

      pltpu.make_async_copy(table_hbm.at[idx_v.at[j1]], buf1, sem1).wait()
      pltpu.sync_copy(buf1, out_hbm.at[pl.ds(base + j1 * _CHUNK, _CHUNK)])

      @pl.when(j1 + 2 < _NCH)
      def _():
        pltpu.make_async_copy(
            table_hbm.at[idx_v.at[j1 + 2]], buf1, sem1).start()

      return ()

    lax.fori_loop(0, _NCH // 2, body, ())

  return gather_kernel


_gather = _make_gather()


def kernel(x, table):
  idx = x.astype(jnp.int32).reshape(_NW, _NCH, _CHUNK)
  out = _gather(idx, table)
  return out.reshape(x.shape[0], x.shape[1], _D)


# speedup vs baseline: 1.0960x; 1.0960x over previous
import functools

import jax
import jax.numpy as jnp
from jax import lax
from jax.experimental import pallas as pl
from jax.experimental.pallas import tpu as pltpu
from jax.experimental.pallas import tpu_sc as plsc

_NUM_CORES = 2
_NUM_SUBCORES = 16
_NW = _NUM_CORES * _NUM_SUBCORES

_CHUNK = 128
_D = 300
_DP = 384
_B = 4096 * 50
_PER_W = _B // _NW
_NCH = _PER_W // _CHUNK


def _make_gather():
  mesh = plsc.VectorSubcoreMesh(core_axis_name="c", subcore_axis_name="s")

  @functools.partial(
      pl.kernel,
      mesh=mesh,
      out_type=jax.ShapeDtypeStruct((_B, _DP), jnp.float32),
      scratch_types=[
          pltpu.VMEM((_NCH, _CHUNK), jnp.int32),
          pltpu.VMEM((_CHUNK, _DP), jnp.float32),
          pltpu.VMEM((_CHUNK, _DP), jnp.float32),
          pltpu.SemaphoreType.DMA,
          pltpu.SemaphoreType.DMA,
      ],
  )
  def gather_kernel(idx_hbm, table_hbm, out_hbm, idx_v, buf0, buf1,
                    sem0, sem1):
    wid = lax.axis_index("s") * _NUM_CORES + lax.axis_index("c")
    base = wid * _PER_W

    pltpu.sync_copy(idx_hbm.at[wid], idx_v)

    pltpu.make_async_copy(table_hbm.at[idx_v.at[0]], buf0, sem0).start()
    pltpu.make_async_copy(table_hbm.at[idx_v.at[1]], buf1, sem1).start()

    def body(i, _):
      j0 = 2 * i
      j1 = 2 * i + 1

      pltpu.make_async_copy(table_hbm.at[idx_v.at[j0]], buf0, sem0).wait()
      pltpu.sync_copy(buf0, out_hbm.at[pl.ds(base + j0 * _CHUNK, _CHUNK)])

      @pl.when(j0 + 2 < _NCH)
      def _():
        pltpu.make_async_copy(
            table_hbm.at[idx_v.at[j0 + 2]], buf0, sem0).start()

      pltpu.make_async_copy(table_hbm.at[idx_v.at[j1]], buf1, sem1).wait()
      pltpu.sync_copy(buf1, out_hbm.at[pl.ds(base + j1 * _CHUNK, _CHUNK)])

      @pl.when(j1 + 2 < _NCH)
      def _():
        pltpu.make_async_copy(
            table_hbm.at[idx_v.at[j1 + 2]], buf1, sem1).start()

      return ()

    lax.fori_loop(0, _NCH // 2, body, ())

  return gather_kernel


_gather = _make_gather()


def kernel(x, table):
  idx = x.astype(jnp.int32).reshape(_NW, _NCH, _CHUNK)
  table_p = jnp.pad(table, ((0, 0), (0, _DP - _D)))
  out = _gather(idx, table_p)
  return out[:, :_D].reshape(x.shape[0], x.shape[1], _D)


# speedup vs baseline: 1.5938x; 1.4541x over previous
import functools

import jax
import jax.numpy as jnp
from jax import lax
from jax.experimental import pallas as pl
from jax.experimental.pallas import tpu as pltpu
from jax.experimental.pallas import tpu_sc as plsc

_NUM_CORES = 2
_NUM_SUBCORES = 16
_NW = _NUM_CORES * _NUM_SUBCORES

_CHUNK = 64
_D = 300
_B = 4096 * 50
_PER_W = _B // _NW
_NCH = _PER_W // _CHUNK
_TAIL_OFF = 172
_TAIL_IN = 256 - _TAIL_OFF


def _make_gather():
  mesh = plsc.VectorSubcoreMesh(core_axis_name="c", subcore_axis_name="s")

  @functools.partial(
      pl.kernel,
      mesh=mesh,
      compiler_params=pltpu.CompilerParams(needs_layout_passes=False),
      out_type=jax.ShapeDtypeStruct((_B, _D), jnp.float32),
      scratch_types=[
          pltpu.VMEM((_NCH, _CHUNK), jnp.int32),
          pltpu.VMEM((_CHUNK, _D), jnp.float32),
          pltpu.VMEM((_CHUNK, _D), jnp.float32),
          pltpu.VMEM((_CHUNK, 128), jnp.float32),
          pltpu.VMEM((_CHUNK, 128), jnp.float32),
          pltpu.SemaphoreType.DMA,
          pltpu.SemaphoreType.DMA,
      ],
  )
  def gather_kernel(idx_hbm, table_hbm, tail_hbm, out_hbm,
                    idx_v, buf0, buf1, tl0, tl1, sem0, sem1):
    wid = lax.axis_index("s") * _NUM_CORES + lax.axis_index("c")
    base = wid * _PER_W

    pltpu.sync_copy(idx_hbm.at[wid], idx_v)

    def start(j, buf, tl, sem):
      idx = idx_v.at[j]
      pltpu.make_async_copy(
          table_hbm.at[idx, pl.ds(0, 128)],
          buf.at[:, pl.ds(0, 128)], sem).start()
      pltpu.make_async_copy(
          table_hbm.at[idx, pl.ds(128, 128)],
          buf.at[:, pl.ds(128, 128)], sem).start()
      pltpu.make_async_copy(tail_hbm.at[idx], tl, sem).start()

    def wait(buf, tl, sem):
      pltpu.make_async_copy(
          table_hbm.at[idx_v.at[0], pl.ds(0, 128)],
          buf.at[:, pl.ds(0, 128)], sem).wait()
      pltpu.make_async_copy(
          table_hbm.at[idx_v.at[0], pl.ds(128, 128)],
          buf.at[:, pl.ds(128, 128)], sem).wait()
      pltpu.make_async_copy(tail_hbm.at[idx_v.at[0]], tl, sem).wait()

    def compact_and_store(j, buf, tl):
      cols = 284 + lax.iota(jnp.int32, 16)
      def row(i, _):
        buf[i, pl.ds(256, 16)] = tl[i, pl.ds(_TAIL_IN, 16)]
        buf[i, pl.ds(272, 16)] = tl[i, pl.ds(_TAIL_IN + 16, 16)]
        rows = jnp.full((16,), i, jnp.int32)
        plsc.store_scatter(buf, [rows, cols], tl[i, pl.ds(_TAIL_IN + 28, 16)])
        return ()
      lax.fori_loop(0, _CHUNK, row, ())
      pltpu.sync_copy(buf, out_hbm.at[pl.ds(base + j * _CHUNK, _CHUNK)])

    start(0, buf0, tl0, sem0)
    start(1, buf1, tl1, sem1)

    def body(i, _):
      j0 = 2 * i
      j1 = 2 * i + 1

      wait(buf0, tl0, sem0)
      compact_and_store(j0, buf0, tl0)

      @pl.when(j0 + 2 < _NCH)
      def _():
        start(j0 + 2, buf0, tl0, sem0)

      wait(buf1, tl1, sem1)
      compact_and_store(j1, buf1, tl1)

      @pl.when(j1 + 2 < _NCH)
      def _():
        start(j1 + 2, buf1, tl1, sem1)

      return ()

    lax.fori_loop(0, _NCH // 2, body, ())

  return gather_kernel


_gather = _make_gather()


def kernel(x, table):
  idx = x.astype(jnp.int32).reshape(_NW, _NCH, _CHUNK)
  tail = table[:, _TAIL_OFF:]
  out = _gather(idx, table, tail)
  return out.reshape(x.shape[0], x.shape[1], _D)
